# baseline (speedup 1.0000x reference)
import jax
import jax.numpy as jnp
from jax.experimental import pallas as pl

N = 10000
E = 320000
D = 128
H = 8
DH = 16
L = 2


def _adapt_body(h_ref, w_ref, b_ref, o_ref):
    o_ref[...] = jax.nn.gelu(h_ref[...] @ w_ref[...] + b_ref[...])


def _adapt(h, W, b):
    BN = 1000
    return pl.pallas_call(
        _adapt_body,
        grid=(N // BN,),
        in_specs=[
            pl.BlockSpec((BN, D), lambda i: (i, 0)),
            pl.BlockSpec((D, D), lambda i: (0, 0)),
            pl.BlockSpec((D,), lambda i: (0,)),
        ],
        out_specs=pl.BlockSpec((BN, D), lambda i: (i, 0)),
        out_shape=jax.ShapeDtypeStruct((N, D), jnp.float32),
    )(h, W, b)


def _hgt_layer(x, src, dst, Wk, Wq, Wv, Wo, ra, rm, pri, skip, lg, lb):
    Wk_eff = jnp.einsum('dhe,hef->dhf', Wk.reshape(D, H, DH), ra) * (pri / jnp.sqrt(DH))[None, :, None]
    Wv_eff = jnp.einsum('dhe,hef->dhf', Wv.reshape(D, H, DH), rm)
    kp = (x @ Wk_eff.reshape(D, D)).reshape(N, H, DH)
    vp = (x @ Wv_eff.reshape(D, D)).reshape(N, H, DH)
    q = (x @ Wq).reshape(N, H, DH)
    k_e = kp[src]
    v_e = vp[src]
    q_e = q[dst]
    ex = jnp.exp(jnp.sum(k_e * q_e, axis=-1))
    denom = jax.ops.segment_sum(ex, dst, num_segments=N)
    agg = jax.ops.segment_sum(v_e * ex[:, :, None], dst, num_segments=N)
    agg = agg / jnp.maximum(denom, 1e-9)[:, :, None]
    trans = jax.nn.gelu(agg.reshape(N, D)) @ Wo
    beta = jax.nn.sigmoid(skip)
    out = trans * beta + x * (1.0 - beta)
    mu = out.mean(axis=-1, keepdims=True)
    var = out.var(axis=-1, keepdims=True)
    return (out - mu) / jnp.sqrt(var + 1e-5) * lg + lb


def kernel(h, edge_index, adapt_W, adapt_b, Wk, Wq, Wv, Wo, rel_att, rel_msg, pri, skip, ln_g, ln_b, out_W, out_b):
    src = edge_index[0]
    dst = edge_index[1]
    x = _adapt(h, adapt_W, adapt_b)
    for i in range(L):
        x = _hgt_layer(x, src, dst, Wk[i], Wq[i], Wv[i], Wo[i], rel_att[i], rel_msg[i], pri[i], skip[i], ln_g[i], ln_b[i])
    return x @ out_W + out_b


# speedup vs baseline: 19.2431x; 19.2431x over previous
import dataclasses

import jax
import jax.numpy as jnp
from jax import lax
from jax.experimental import pallas as pl
from jax.experimental.pallas import tpu as pltpu
from jax.experimental.pallas import tpu_sc as plsc

N = 10000
E = 320000
D = 128
H = 8
DH = 16
L = 2

NC = 2
NS = 16
EC = E // (NC * NS)
CHUNK = 40
NCHUNK = EC // CHUNK
ZROW = 40
NCH_Z = N // ZROW
ZITER = (NCH_Z + NS - 1) // NS
AW = H * DH + DH


def _adapt_body(h_ref, w_ref, b_ref, o_ref):
    o_ref[...] = jax.nn.gelu(h_ref[...] @ w_ref[...] + b_ref[...])


def _adapt(h, W, b):
    BN = 1000
    return pl.pallas_call(
        _adapt_body,
        grid=(N // BN,),
        in_specs=[
            pl.BlockSpec((BN, D), lambda i: (i, 0)),
            pl.BlockSpec((D, D), lambda i: (0, 0)),
            pl.BlockSpec((D,), lambda i: (0,)),
        ],
        out_specs=pl.BlockSpec((BN, D), lambda i: (i, 0)),
        out_shape=jax.ShapeDtypeStruct((N, D), jnp.float32),
    )(h, W, b)


def _proj_body(x_ref, wkv_ref, wq_ref, kv_ref, q_ref):
    x = x_ref[...]
    kv_ref[...] = x @ wkv_ref[...]
    q_ref[...] = x @ wq_ref[...]


def _proj(x, Wkv, Wq):
    BN = 1000
    return pl.pallas_call(
        _proj_body,
        grid=(N // BN,),
        in_specs=[
            pl.BlockSpec((BN, D), lambda i: (i, 0)),
            pl.BlockSpec((D, 2 * D), lambda i: (0, 0)),
            pl.BlockSpec((D, D), lambda i: (0, 0)),
        ],
        out_specs=[
            pl.BlockSpec((BN, 2 * D), lambda i: (i, 0)),
            pl.BlockSpec((BN, D), lambda i: (i, 0)),
        ],
        out_shape=[
            jax.ShapeDtypeStruct((N, 2 * D), jnp.float32),
            jax.ShapeDtypeStruct((N, D), jnp.float32),
        ],
    )(x, Wkv, Wq)


def _post_body(a0_ref, a1_ref, x_ref, wo_ref, rep_ref, lg_ref, lb_ref, c_ref, o_ref):
    a0 = a0_ref[...]
    a1 = a1_ref[...]
    agg = a0[:, :D] + a1[:, :D]
    den8 = a0[:, D:D + H] + a1[:, D:D + H]
    den = jnp.maximum(den8 @ rep_ref[...], 1e-9)
    t = jax.nn.gelu(agg / den) @ wo_ref[...]
    out = t + x_ref[...] * c_ref[0, 0]
    mu = jnp.mean(out, axis=-1, keepdims=True)
    var = jnp.mean((out - mu) ** 2, axis=-1, keepdims=True)
    o_ref[...] = (out - mu) * jax.lax.rsqrt(var + 1e-5) * lg_ref[...] + lb_ref[...]


def _post(a0, a1, x, Wo_beta, rep, lg, lb, c):
    BN = 1000
    return pl.pallas_call(
        _post_body,
        grid=(N // BN,),
        in_specs=[
            pl.BlockSpec((BN, AW), lambda i: (i, 0)),
            pl.BlockSpec((BN, AW), lambda i: (i, 0)),
            pl.BlockSpec((BN, D), lambda i: (i, 0)),
            pl.BlockSpec((D, D), lambda i: (0, 0)),
            pl.BlockSpec((H, D), lambda i: (0, 0)),
            pl.BlockSpec((1, D), lambda i: (0, 0)),
            pl.BlockSpec((1, D), lambda i: (0, 0)),
            pl.BlockSpec((1, 1), lambda i: (0, 0)),
        ],
        out_specs=pl.BlockSpec((BN, D), lambda i: (i, 0)),
        out_shape=jax.ShapeDtypeStruct((N, D), jnp.float32),
    )(a0, a1, x, Wo_beta, rep, lg, lb, c)


def _final_body(x_ref, w_ref, b_ref, o_ref):
    o_ref[...] = x_ref[...] @ w_ref[...] + b_ref[...]


def _final(x, W, b):
    BN = 1000
    return pl.pallas_call(
        _final_body,
        grid=(N // BN,),
        in_specs=[
            pl.BlockSpec((BN, D), lambda i: (i, 0)),
            pl.BlockSpec((D, D), lambda i: (0, 0)),
            pl.BlockSpec((D,), lambda i: (0,)),
        ],
        out_specs=pl.BlockSpec((BN, D), lambda i: (i, 0)),
        out_shape=jax.ShapeDtypeStruct((N, D), jnp.float32),
    )(x, W, b)


def _edge_body(kv_hbm, q_hbm, src_hbm, dst_hbm, out_hbm,
               isrc, idst, kv_v, q_v, out_v, zbuf, acc, sem_kv, sem_q):
    c = lax.axis_index("c")
    s = lax.axis_index("s")
    lanes = lax.iota(jnp.int32, 16)

    @pl.loop(0, ZROW)
    def _(r):
        @pl.loop(0, AW // 16)
        def _(g):
            zbuf[r, pl.ds(g * 16, 16)] = jnp.zeros((16,), jnp.float32)

    @pl.loop(0, ZITER)
    def _(t):
        cid = s + NS * t

        @pl.when(cid < NCH_Z)
        def _():
            pltpu.sync_copy(zbuf, acc.at[pl.ds(cid * ZROW, ZROW)])

    plsc.subcore_barrier()

    base = c * (NS * EC) + s * EC

    @pl.loop(0, NCHUNK)
    def _(i):
        e0 = base + i * CHUNK
        pltpu.sync_copy(src_hbm.at[pl.ds(e0, CHUNK)], isrc)
        pltpu.sync_copy(dst_hbm.at[pl.ds(e0, CHUNK)], idst)
        cp_kv = pltpu.async_copy(kv_hbm.at[isrc], kv_v, sem_kv)
        cp_q = pltpu.async_copy(q_hbm.at[idst], q_v, sem_q)
        cp_kv.wait()
        cp_q.wait()

        @pl.loop(0, CHUNK)
        def _(e):
            exrow = jnp.zeros((16,), jnp.float32)
            for h in range(H):
                kvec = kv_v[e, pl.ds(h * DH, DH)]
                qvec = q_v[e, pl.ds(h * DH, DH)]
                att = jnp.sum(kvec * qvec)
                exv = jnp.exp(jnp.full((16,), att, jnp.float32))
                vvec = kv_v[e, pl.ds(D + h * DH, DH)]
                out_v[e, pl.ds(h * DH, DH)] = vvec * exv
                exrow = jnp.where(lanes == h, exv, exrow)
            out_v[e, pl.ds(D, DH)] = exrow

        pltpu.sync_copy(out_v, acc.at[idst], add=True)

    plsc.subcore_barrier()

    @pl.loop(0, ZITER)
    def _(t):
        cid = s + NS * t

        @pl.when(cid < NCH_Z)
        def _():
            r0 = cid * ZROW
            pltpu.sync_copy(acc.at[pl.ds(r0, ZROW)],
                            out_hbm.at[pl.ds(c * N + r0, ZROW)])


def _edge_phase(kv_tab, q_tab, src, dst):
    mesh = plsc.VectorSubcoreMesh(core_axis_name="c", subcore_axis_name="s")
    cp = pltpu.CompilerParams()
    if "needs_layout_passes" in pltpu.CompilerParams.__dataclass_fields__:
        cp = dataclasses.replace(cp, needs_layout_passes=False)
    cp = dataclasses.replace(cp, use_tc_tiling_on_sc=False)
    k = pl.kernel(
        _edge_body,
        out_type=jax.ShapeDtypeStruct((NC * N, AW), jnp.float32),
        mesh=mesh,
        scratch_types=[
            pltpu.VMEM((CHUNK,), jnp.int32),
            pltpu.VMEM((CHUNK,), jnp.int32),
            pltpu.VMEM((CHUNK, 2 * D), jnp.float32),
            pltpu.VMEM((CHUNK, D), jnp.float32),
            pltpu.VMEM((CHUNK, AW), jnp.float32),
            pltpu.VMEM((ZROW, AW), jnp.float32),
            pltpu.VMEM_SHARED((N, AW), jnp.float32),
            pltpu.SemaphoreType.DMA,
            pltpu.SemaphoreType.DMA,
        ],
        compiler_params=cp,
    )
    return k(kv_tab, q_tab, src, dst)


def _hgt_layer(x, src, dst, Wk, Wq, Wv, Wo, ra, rm, pri, skip, lg, lb, rep):
    scale = (pri / jnp.sqrt(jnp.float32(DH)))[None, :, None]
    Wk_eff = (jnp.einsum('dhe,hef->dhf', Wk.reshape(D, H, DH), ra) * scale).reshape(D, D)
    Wv_eff = jnp.einsum('dhe,hef->dhf', Wv.reshape(D, H, DH), rm).reshape(D, D)
    Wkv = jnp.concatenate([Wk_eff, Wv_eff], axis=1)
    beta = jax.nn.sigmoid(skip)
    Wo_beta = Wo * beta
    c = (1.0 - beta).reshape(1, 1)

    kv_tab, q_tab = _proj(x, Wkv, Wq)
    parts = _edge_phase(kv_tab, q_tab, src, dst)
    return _post(parts[:N], parts[N:], x, Wo_beta, rep,
                 lg.reshape(1, D), lb.reshape(1, D), c)


def kernel(h, edge_index, adapt_W, adapt_b, Wk, Wq, Wv, Wo, rel_att, rel_msg, pri, skip, ln_g, ln_b, out_W, out_b):
    src = edge_index[0]
    dst = edge_index[1]
    rep = jnp.repeat(jnp.eye(H, dtype=jnp.float32), DH, axis=1)
    x = _adapt(h, adapt_W, adapt_b)
    for i in range(L):
        x = _hgt_layer(x, src, dst, Wk[i], Wq[i], Wv[i], Wo[i], rel_att[i],
                       rel_msg[i], pri[i], skip[i], ln_g[i], ln_b[i], rep)
    return _final(x, out_W, out_b)


# speedup vs baseline: 21.8037x; 1.1331x over previous
import dataclasses

import jax
import jax.numpy as jnp
from jax import lax
from jax.experimental import pallas as pl
from jax.experimental.pallas import tpu as pltpu
from jax.experimental.pallas import tpu_sc as plsc

N = 10000
E = 320000
D = 128
H = 8
DH = 16
L = 2

NC = 2
NS = 16
EC = E // (NC * NS)
CHUNK = 40
NCHUNK = EC // CHUNK
ZROW = 16
NCH_Z = N // ZROW
ZITER = (NCH_Z + NS - 1) // NS
AW = H * DH + DH


def _adapt_body(h_ref, w_ref, b_ref, o_ref):
    o_ref[...] = jax.nn.gelu(h_ref[...] @ w_ref[...] + b_ref[...])


def _adapt(h, W, b):
    BN = 1000
    return pl.pallas_call(
        _adapt_body,
        grid=(N // BN,),
        in_specs=[
            pl.BlockSpec((BN, D), lambda i: (i, 0)),
            pl.BlockSpec((D, D), lambda i: (0, 0)),
            pl.BlockSpec((D,), lambda i: (0,)),
        ],
        out_specs=pl.BlockSpec((BN, D), lambda i: (i, 0)),
        out_shape=jax.ShapeDtypeStruct((N, D), jnp.float32),
    )(h, W, b)


def _proj_body(x_ref, wkv_ref, wq_ref, kv_ref, q_ref):
    x = x_ref[...]
    kv_ref[...] = x @ wkv_ref[...]
    q_ref[...] = x @ wq_ref[...]


def _proj(x, Wkv, Wq):
    BN = 1000
    return pl.pallas_call(
        _proj_body,
        grid=(N // BN,),
        in_specs=[
            pl.BlockSpec((BN, D), lambda i: (i, 0)),
            pl.BlockSpec((D, 2 * D), lambda i: (0, 0)),
            pl.BlockSpec((D, D), lambda i: (0, 0)),
        ],
        out_specs=[
            pl.BlockSpec((BN, 2 * D), lambda i: (i, 0)),
            pl.BlockSpec((BN, D), lambda i: (i, 0)),
        ],
        out_shape=[
            jax.ShapeDtypeStruct((N, 2 * D), jnp.float32),
            jax.ShapeDtypeStruct((N, D), jnp.float32),
        ],
    )(x, Wkv, Wq)


def _post_body(a0_ref, a1_ref, x_ref, wo_ref, rep_ref, lg_ref, lb_ref, c_ref, o_ref):
    a0 = a0_ref[...]
    a1 = a1_ref[...]
    agg = a0[:, :D] + a1[:, :D]
    den8 = a0[:, D:D + H] + a1[:, D:D + H]
    den = jnp.maximum(den8 @ rep_ref[...], 1e-9)
    t = jax.nn.gelu(agg / den) @ wo_ref[...]
    out = t + x_ref[...] * c_ref[0, 0]
    mu = jnp.mean(out, axis=-1, keepdims=True)
    var = jnp.mean((out - mu) ** 2, axis=-1, keepdims=True)
    o_ref[...] = (out - mu) * jax.lax.rsqrt(var + 1e-5) * lg_ref[...] + lb_ref[...]


def _post(a0, a1, x, Wo_beta, rep, lg, lb, c):
    BN = 1000
    return pl.pallas_call(
        _post_body,
        grid=(N // BN,),
        in_specs=[
            pl.BlockSpec((BN, AW), lambda i: (i, 0)),
            pl.BlockSpec((BN, AW), lambda i: (i, 0)),
            pl.BlockSpec((BN, D), lambda i: (i, 0)),
            pl.BlockSpec((D, D), lambda i: (0, 0)),
            pl.BlockSpec((H, D), lambda i: (0, 0)),
            pl.BlockSpec((1, D), lambda i: (0, 0)),
            pl.BlockSpec((1, D), lambda i: (0, 0)),
            pl.BlockSpec((1, 1), lambda i: (0, 0)),
        ],
        out_specs=pl.BlockSpec((BN, D), lambda i: (i, 0)),
        out_shape=jax.ShapeDtypeStruct((N, D), jnp.float32),
    )(a0, a1, x, Wo_beta, rep, lg, lb, c)


def _final_body(x_ref, w_ref, b_ref, o_ref):
    o_ref[...] = x_ref[...] @ w_ref[...] + b_ref[...]


def _final(x, W, b):
    BN = 1000
    return pl.pallas_call(
        _final_body,
        grid=(N // BN,),
        in_specs=[
            pl.BlockSpec((BN, D), lambda i: (i, 0)),
            pl.BlockSpec((D, D), lambda i: (0, 0)),
            pl.BlockSpec((D,), lambda i: (0,)),
        ],
        out_specs=pl.BlockSpec((BN, D), lambda i: (i, 0)),
        out_shape=jax.ShapeDtypeStruct((N, D), jnp.float32),
    )(x, W, b)


def _edge_body(kv_hbm, q_hbm, src_hbm, dst_hbm, out_hbm,
               isrc_a, idst_a, kv_a, q_a, isrc_b, idst_b, kv_b, q_b,
               out_v, zbuf, acc,
               sem_kv_a, sem_q_a, sem_kv_b, sem_q_b):
    c = lax.axis_index("c")
    s = lax.axis_index("s")
    lanes = lax.iota(jnp.int32, 16)

    @pl.loop(0, ZROW)
    def _(r):
        @pl.loop(0, AW // 16)
        def _(g):
            zbuf[r, pl.ds(g * 16, 16)] = jnp.zeros((16,), jnp.float32)

    @pl.loop(0, ZITER)
    def _(t):
        cid = s + NS * t

        @pl.when(cid < NCH_Z)
        def _():
            pltpu.sync_copy(zbuf, acc.at[pl.ds(cid * ZROW, ZROW)])

    plsc.subcore_barrier()

    base = c * (NS * EC) + s * EC

    def fetch(i, isrc, idst, kv_v, q_v, sem_kv, sem_q):
        e0 = base + i * CHUNK
        pltpu.sync_copy(src_hbm.at[pl.ds(e0, CHUNK)], isrc)
        pltpu.sync_copy(dst_hbm.at[pl.ds(e0, CHUNK)], idst)
        pltpu.async_copy(kv_hbm.at[isrc], kv_v, sem_kv)
        pltpu.async_copy(q_hbm.at[idst], q_v, sem_q)

    def process(isrc, idst, kv_v, q_v, sem_kv, sem_q):
        pltpu.make_async_copy(kv_hbm.at[isrc], kv_v, sem_kv).wait()
        pltpu.make_async_copy(q_hbm.at[idst], q_v, sem_q).wait()

        @pl.loop(0, CHUNK)
        def _(e):
            exrow = jnp.zeros((16,), jnp.float32)
            for h in range(H):
                kvec = kv_v[e, pl.ds(h * DH, DH)]
                qvec = q_v[e, pl.ds(h * DH, DH)]
                att = jnp.sum(kvec * qvec)
                exv = jnp.exp(jnp.full((16,), att, jnp.float32))
                vvec = kv_v[e, pl.ds(D + h * DH, DH)]
                out_v[e, pl.ds(h * DH, DH)] = vvec * exv
                exrow = jnp.where(lanes == h, exv, exrow)
            out_v[e, pl.ds(D, DH)] = exrow

        pltpu.sync_copy(out_v, acc.at[idst], add=True)

    fetch(0, isrc_a, idst_a, kv_a, q_a, sem_kv_a, sem_q_a)

    @pl.loop(0, NCHUNK // 2)
    def _(j):
        c0 = 2 * j
        fetch(c0 + 1, isrc_b, idst_b, kv_b, q_b, sem_kv_b, sem_q_b)
        process(isrc_a, idst_a, kv_a, q_a, sem_kv_a, sem_q_a)

        @pl.when(c0 + 2 < NCHUNK)
        def _():
            fetch(c0 + 2, isrc_a, idst_a, kv_a, q_a, sem_kv_a, sem_q_a)

        process(isrc_b, idst_b, kv_b, q_b, sem_kv_b, sem_q_b)

    plsc.subcore_barrier()

    @pl.loop(0, ZITER)
    def _(t):
        cid = s + NS * t

        @pl.when(cid < NCH_Z)
        def _():
            r0 = cid * ZROW
            pltpu.sync_copy(acc.at[pl.ds(r0, ZROW)],
                            out_hbm.at[pl.ds(c * N + r0, ZROW)])


def _edge_phase(kv_tab, q_tab, src, dst):
    mesh = plsc.VectorSubcoreMesh(core_axis_name="c", subcore_axis_name="s")
    cp = pltpu.CompilerParams()
    if "needs_layout_passes" in pltpu.CompilerParams.__dataclass_fields__:
        cp = dataclasses.replace(cp, needs_layout_passes=False)
    cp = dataclasses.replace(cp, use_tc_tiling_on_sc=False)
    k = pl.kernel(
        _edge_body,
        out_type=jax.ShapeDtypeStruct((NC * N, AW), jnp.float32),
        mesh=mesh,
        scratch_types=[
            pltpu.VMEM((CHUNK,), jnp.int32),
            pltpu.VMEM((CHUNK,), jnp.int32),
            pltpu.VMEM((CHUNK, 2 * D), jnp.float32),
            pltpu.VMEM((CHUNK, D), jnp.float32),
            pltpu.VMEM((CHUNK,), jnp.int32),
            pltpu.VMEM((CHUNK,), jnp.int32),
            pltpu.VMEM((CHUNK, 2 * D), jnp.float32),
            pltpu.VMEM((CHUNK, D), jnp.float32),
            pltpu.VMEM((CHUNK, AW), jnp.float32),
            pltpu.VMEM((ZROW, AW), jnp.float32),
            pltpu.VMEM_SHARED((N, AW), jnp.float32),
            pltpu.SemaphoreType.DMA,
            pltpu.SemaphoreType.DMA,
            pltpu.SemaphoreType.DMA,
            pltpu.SemaphoreType.DMA,
        ],
        compiler_params=cp,
    )
    return k(kv_tab, q_tab, src, dst)


def _hgt_layer(x, src, dst, Wk, Wq, Wv, Wo, ra, rm, pri, skip, lg, lb, rep):
    scale = (pri / jnp.sqrt(jnp.float32(DH)))[None, :, None]
    Wk_eff = (jnp.einsum('dhe,hef->dhf', Wk.reshape(D, H, DH), ra) * scale).reshape(D, D)
    Wv_eff = jnp.einsum('dhe,hef->dhf', Wv.reshape(D, H, DH), rm).reshape(D, D)
    Wkv = jnp.concatenate([Wk_eff, Wv_eff], axis=1)
    beta = jax.nn.sigmoid(skip)
    Wo_beta = Wo * beta
    c = (1.0 - beta).reshape(1, 1)

    kv_tab, q_tab = _proj(x, Wkv, Wq)
    parts = _edge_phase(kv_tab, q_tab, src, dst)
    return _post(parts[:N], parts[N:], x, Wo_beta, rep,
                 lg.reshape(1, D), lb.reshape(1, D), c)


def kernel(h, edge_index, adapt_W, adapt_b, Wk, Wq, Wv, Wo, rel_att, rel_msg, pri, skip, ln_g, ln_b, out_W, out_b):
    src = edge_index[0]
    dst = edge_index[1]
    rep = jnp.repeat(jnp.eye(H, dtype=jnp.float32), DH, axis=1)
    x = _adapt(h, adapt_W, adapt_b)
    for i in range(L):
        x = _hgt_layer(x, src, dst, Wk[i], Wq[i], Wv[i], Wo[i], rel_att[i],
                       rel_msg[i], pri[i], skip[i], ln_g[i], ln_b[i], rep)
    return _final(x, out_W, out_b)
